# baseline (speedup 1.0000x reference)
import functools

import jax
import jax.numpy as jnp
from jax import lax
from jax.experimental import pallas as pl
from jax.experimental.pallas import tpu as pltpu
from jax.experimental.pallas import tpu_sc as plsc

_VOCAB = 1_000_000
_EMBED = 32
_BATCH = 16384
_FIELDS = 26

_info = plsc.get_sparse_core_info()
_NC = _info.num_cores
_NS = _info.num_subcores
_NW = _NC * _NS
_BPW = _BATCH // _NW
_L = 16

_mesh = plsc.VectorSubcoreMesh(core_axis_name="c", subcore_axis_name="s")


_TC = 512
_NCHUNK = _VOCAB // _TC
_NFULL = (_NCHUNK // _NW) * _NW
_CPW = _NFULL // _NW
_TAIL = _VOCAB - _NFULL * _TC

_DIAG = [
    tuple((d + j) % _EMBED for j in range(_L)) for d in range(_EMBED)
]


@functools.partial(
    pl.kernel,
    mesh=_mesh,
    out_type=jax.ShapeDtypeStruct((_VOCAB, _EMBED), jnp.float32),
    scratch_types=[
        pltpu.VMEM((_EMBED, _TC), jnp.float32),
        pltpu.VMEM((_EMBED, _TC), jnp.float32),
        pltpu.VMEM((_TC, _EMBED), jnp.float32),
        pltpu.VMEM((_TC, _EMBED), jnp.float32),
        pltpu.SemaphoreType.DMA,
        pltpu.SemaphoreType.DMA,
        pltpu.SemaphoreType.DMA,
        pltpu.SemaphoreType.DMA,
    ],
    compiler_params=pltpu.CompilerParams(
        use_tc_tiling_on_sc=False, needs_layout_passes=False),
)
def _transpose_table(wt_hbm, out_hbm, s0, s1, d0, d1,
                     gsem0, gsem1, wsem0, wsem1):
    wid = lax.axis_index("s") * _NC + lax.axis_index("c")
    sbufs = (s0, s1)
    dbufs = (d0, d1)
    gsems = (gsem0, gsem1)
    wsems = (wsem0, wsem1)
    iota = lax.iota(jnp.int32, _L)

    def v0_of(c):
        return (wid + _NW * c) * _TC

    def gdesc(c, p):
        return pltpu.make_async_copy(
            wt_hbm.at[:, pl.ds(v0_of(c), _TC)], sbufs[p], gsems[p])

    def wdesc(c, p):
        return pltpu.make_async_copy(
            dbufs[p], out_hbm.at[pl.ds(v0_of(c), _TC)], wsems[p])

    diags = [(iota + dd) & (_EMBED - 1) for dd in range(_EMBED)]

    def transpose_block(s, d, ngroups):
        @plsc.parallel_loop(0, ngroups, unroll=2)
        def _(g):
            cols = g * _L + iota
            for dd in range(_EMBED):
                vals = plsc.load_gather(s, [diags[dd], cols])
                plsc.store_scatter(d, [cols, diags[dd]], vals)

    def step(c, p):
        @pl.when(c + 1 < _CPW)
        def _():
            gdesc(jnp.minimum(c + 1, _CPW - 1), 1 - p).start()

        gdesc(c, p).wait()

        @pl.when(c >= 2)
        def _():
            wdesc(c - 2, p).wait()

        transpose_block(sbufs[p], dbufs[p], _TC // _L)
        wdesc(c, p).start()

    gdesc(0, 0).start()

    def body(k, carry):
        step(2 * k, 0)
        step(2 * k + 1, 1)
        return carry

    lax.fori_loop(0, _CPW // 2, body, 0)
    step(_CPW - 1, 0)
    wdesc(_CPW - 2, 1).wait()
    wdesc(_CPW - 1, 0).wait()

    @pl.when(wid == _NW - 1)
    def _():
        for tv0 in (_NFULL * _TC, _VOCAB - _TC):
            pltpu.sync_copy(wt_hbm.at[:, pl.ds(tv0, _TC)], s0)
            transpose_block(s0, d0, _TC // _L)
            pltpu.sync_copy(d0, out_hbm.at[pl.ds(tv0, _TC)])


@functools.partial(
    pl.kernel,
    mesh=_mesh,
    out_type=jax.ShapeDtypeStruct((_FIELDS, _EMBED, _BATCH), jnp.float32),
    scratch_types=[
        pltpu.VMEM((_FIELDS, _BPW), jnp.int32),
        pltpu.VMEM((_BPW, _EMBED), jnp.float32),
        pltpu.VMEM((_BPW, _EMBED), jnp.float32),
        pltpu.VMEM((_EMBED, _BPW + 1), jnp.float32),
        pltpu.VMEM((_EMBED, _BPW + 1), jnp.float32),
        pltpu.SemaphoreType.DMA,
        pltpu.SemaphoreType.DMA,
        pltpu.SemaphoreType.DMA,
        pltpu.SemaphoreType.DMA,
    ],
    compiler_params=pltpu.CompilerParams(
        use_tc_tiling_on_sc=False, needs_layout_passes=False),
)
def _gather_all(table_hbm, idx_hbm, out_hbm, idx_v, r0, r1, t0buf, t1buf,
                gsem0, gsem1, wsem0, wsem1):
    wid = lax.axis_index("s") * _NC + lax.axis_index("c")
    b0 = wid * _BPW
    rbufs = (r0, r1)
    tbufs = (t0buf, t1buf)
    gsems = (gsem0, gsem1)
    wsems = (wsem0, wsem1)

    pltpu.sync_copy(idx_hbm.at[:, pl.ds(b0, _BPW)], idx_v)

    def gdesc(f, p):
        return pltpu.make_async_copy(
            table_hbm.at[idx_v.at[f]], rbufs[p], gsems[p])

    def wdesc(f, p):
        return pltpu.make_async_copy(
            tbufs[p].at[:, pl.ds(0, _BPW)],
            out_hbm.at[f, :, pl.ds(b0, _BPW)], wsems[p])

    iota = lax.iota(jnp.int32, _L)
    rows0 = iota
    rows1 = iota + _L

    def transpose_chunk(r, t):
        @plsc.parallel_loop(0, _BPW, unroll=4)
        def _(i):
            ci = jnp.full((_L,), i, jnp.int32)
            v0 = r[i, pl.ds(0, _L)]
            v1 = r[i, pl.ds(_L, _L)]
            plsc.store_scatter(t, [rows0, ci], v0)
            plsc.store_scatter(t, [rows1, ci], v1)

    def step(f, p):
        fn = jnp.minimum(f + 1, _FIELDS - 1)

        @pl.when(f + 1 < _FIELDS)
        def _():
            gdesc(fn, 1 - p).start()

        gdesc(f, p).wait()

        @pl.when(f >= 2)
        def _():
            wdesc(f - 2, p).wait()

        transpose_chunk(rbufs[p], tbufs[p])
        wdesc(f, p).start()

    gdesc(0, 0).start()

    def body(k, carry):
        step(2 * k, 0)
        step(2 * k + 1, 1)
        return carry

    lax.fori_loop(0, _FIELDS // 2, body, 0)
    wdesc(_FIELDS - 2, 0).wait()
    wdesc(_FIELDS - 1, 1).wait()


def kernel(tokenIndex, weights):
    w_rm = _transpose_table(weights.T)
    idx_t = tokenIndex.T.astype(jnp.int32)
    out_t = _gather_all(w_rm, idx_t)
    return out_t.transpose(2, 0, 1)


# speedup vs baseline: 2.0669x; 2.0669x over previous
import functools

import jax
import jax.numpy as jnp
from jax import lax
from jax.experimental import pallas as pl
from jax.experimental.pallas import tpu as pltpu
from jax.experimental.pallas import tpu_sc as plsc

_VOCAB = 1_000_000
_EMBED = 32
_HALF = _EMBED // 2
_BATCH = 16384
_FIELDS = 26

_info = plsc.get_sparse_core_info()
_NC = _info.num_cores
_NS = _info.num_subcores
_NW = _NC * _NS
_BPW = _BATCH // _NW
_L = 16

_mesh = plsc.VectorSubcoreMesh(core_axis_name="c", subcore_axis_name="s")


@functools.partial(
    pl.kernel,
    mesh=_mesh,
    out_type=jax.ShapeDtypeStruct((_FIELDS, _EMBED, _BATCH), jnp.float32),
    scratch_types=[
        pltpu.VMEM((_FIELDS, _BPW), jnp.int32),
        pltpu.VMEM((_BPW, _HALF), jnp.float32),
        pltpu.VMEM((_BPW, _HALF), jnp.float32),
        pltpu.VMEM((_BPW, _HALF), jnp.float32),
        pltpu.VMEM((_BPW, _HALF), jnp.float32),
        pltpu.VMEM((_EMBED, _BPW + 1), jnp.float32),
        pltpu.VMEM((_EMBED, _BPW + 1), jnp.float32),
        pltpu.SemaphoreType.DMA,
        pltpu.SemaphoreType.DMA,
        pltpu.SemaphoreType.DMA,
        pltpu.SemaphoreType.DMA,
        pltpu.SemaphoreType.DMA,
        pltpu.SemaphoreType.DMA,
    ],
    compiler_params=pltpu.CompilerParams(
        use_tc_tiling_on_sc=False, needs_layout_passes=False),
)
def _gather_all(tab_a, tab_b, idx_hbm, out_hbm, idx_v,
                ra0, ra1, rb0, rb1, t0buf, t1buf,
                gsa0, gsa1, gsb0, gsb1, wsem0, wsem1):
    wid = lax.axis_index("s") * _NC + lax.axis_index("c")
    b0 = wid * _BPW
    rabufs = (ra0, ra1)
    rbbufs = (rb0, rb1)
    tbufs = (t0buf, t1buf)
    gasems = (gsa0, gsa1)
    gbsems = (gsb0, gsb1)
    wsems = (wsem0, wsem1)

    pltpu.sync_copy(idx_hbm.at[:, pl.ds(b0, _BPW)], idx_v)

    def gdesc_a(f, p):
        return pltpu.make_async_copy(
            tab_a.at[idx_v.at[f]], rabufs[p], gasems[p])

    def gdesc_b(f, p):
        return pltpu.make_async_copy(
            tab_b.at[idx_v.at[f]], rbbufs[p], gbsems[p])

    def wdesc(f, p):
        return pltpu.make_async_copy(
            tbufs[p].at[:, pl.ds(0, _BPW)],
            out_hbm.at[f, :, pl.ds(b0, _BPW)], wsems[p])

    iota = lax.iota(jnp.int32, _L)
    rows0 = iota
    rows1 = iota + _L

    def transpose_chunk(ra, rb, t):
        @plsc.parallel_loop(0, _BPW, unroll=4)
        def _(i):
            ci = jnp.full((_L,), i, jnp.int32)
            v0 = ra[i, pl.ds(0, _L)]
            v1 = rb[i, pl.ds(0, _L)]
            plsc.store_scatter(t, [rows0, ci], v0)
            plsc.store_scatter(t, [rows1, ci], v1)

    def step(f, p):
        fn = jnp.minimum(f + 1, _FIELDS - 1)

        @pl.when(f + 1 < _FIELDS)
        def _():
            gdesc_a(fn, 1 - p).start()
            gdesc_b(fn, 1 - p).start()

        gdesc_a(f, p).wait()
        gdesc_b(f, p).wait()

        @pl.when(f >= 2)
        def _():
            wdesc(f - 2, p).wait()

        transpose_chunk(rabufs[p], rbbufs[p], tbufs[p])
        wdesc(f, p).start()

    gdesc_a(0, 0).start()
    gdesc_b(0, 0).start()

    def body(k, carry):
        step(2 * k, 0)
        step(2 * k + 1, 1)
        return carry

    lax.fori_loop(0, _FIELDS // 2, body, 0)
    wdesc(_FIELDS - 2, 0).wait()
    wdesc(_FIELDS - 1, 1).wait()


def kernel(tokenIndex, weights):
    w_a = weights[:, :_HALF]
    w_b = weights[:, _HALF:]
    idx_t = tokenIndex.T.astype(jnp.int32)
    out_t = _gather_all(w_a, w_b, idx_t)
    return out_t.transpose(2, 0, 1)


# speedup vs baseline: 4.5279x; 2.1907x over previous
import functools

import jax
import jax.numpy as jnp
from jax import lax
from jax.experimental import pallas as pl
from jax.experimental.pallas import tpu as pltpu
from jax.experimental.pallas import tpu_sc as plsc

_VOCAB = 1_000_000
_EMBED = 32
_BATCH = 16384
_FIELDS = 26

_info = plsc.get_sparse_core_info()
_NC = _info.num_cores
_NS = _info.num_subcores
_NW = _NC * _NS
_BPW = _BATCH // _NW
_L = 16

_mesh = plsc.VectorSubcoreMesh(core_axis_name="c", subcore_axis_name="s")


@functools.partial(
    pl.kernel,
    mesh=_mesh,
    out_type=jax.ShapeDtypeStruct((_FIELDS, _EMBED, _BATCH), jnp.float32),
    scratch_types=[
        pltpu.VMEM((_FIELDS, _BPW), jnp.int32),
        pltpu.VMEM((_BPW, _EMBED), jnp.float32),
        pltpu.VMEM((_BPW, _EMBED), jnp.float32),
        pltpu.VMEM((_EMBED, _BPW + 1), jnp.float32),
        pltpu.VMEM((_EMBED, _BPW + 1), jnp.float32),
        pltpu.SemaphoreType.DMA,
        pltpu.SemaphoreType.DMA,
        pltpu.SemaphoreType.DMA,
        pltpu.SemaphoreType.DMA,
    ],
    compiler_params=pltpu.CompilerParams(
        use_tc_tiling_on_sc=False, needs_layout_passes=False),
)
def _gather_all(table_hbm, idx_hbm, out_hbm, idx_v, r0, r1, t0buf, t1buf,
                gsem0, gsem1, wsem0, wsem1):
    wid = lax.axis_index("s") * _NC + lax.axis_index("c")
    b0 = wid * _BPW
    rbufs = (r0, r1)
    tbufs = (t0buf, t1buf)
    gsems = (gsem0, gsem1)
    wsems = (wsem0, wsem1)

    pltpu.sync_copy(idx_hbm.at[:, pl.ds(b0, _BPW)], idx_v)

    def gdesc(f, p):
        return pltpu.make_async_copy(
            table_hbm.at[idx_v.at[f]], rbufs[p], gsems[p])

    def wdesc(f, p):
        return pltpu.make_async_copy(
            tbufs[p].at[:, pl.ds(0, _BPW)],
            out_hbm.at[f, :, pl.ds(b0, _BPW)], wsems[p])

    iota = lax.iota(jnp.int32, _L)
    rows0 = iota
    rows1 = iota + _L

    def transpose_chunk(r, t):
        @plsc.parallel_loop(0, _BPW, unroll=4)
        def _(i):
            ci = jnp.full((_L,), i, jnp.int32)
            v0 = r[i, pl.ds(0, _L)]
            v1 = r[i, pl.ds(_L, _L)]
            plsc.store_scatter(t, [rows0, ci], v0)
            plsc.store_scatter(t, [rows1, ci], v1)

    def step(f, p):
        fn = jnp.minimum(f + 1, _FIELDS - 1)

        @pl.when(f + 1 < _FIELDS)
        def _():
            gdesc(fn, 1 - p).start()

        gdesc(f, p).wait()

        @pl.when(f >= 2)
        def _():
            wdesc(f - 2, p).wait()

        transpose_chunk(rbufs[p], tbufs[p])
        wdesc(f, p).start()

    gdesc(0, 0).start()

    def body(k, carry):
        step(2 * k, 0)
        step(2 * k + 1, 1)
        return carry

    lax.fori_loop(0, _FIELDS // 2, body, 0)
    wdesc(_FIELDS - 2, 0).wait()
    wdesc(_FIELDS - 1, 1).wait()


def kernel(tokenIndex, weights):
    idx_t = tokenIndex.T.astype(jnp.int32)
    out_t = _gather_all(weights, idx_t)
    return out_t.transpose(2, 0, 1)
